# baseline (speedup 1.0000x reference)
import functools

import jax
import jax.numpy as jnp
from jax.experimental import pallas as pl
from jax.experimental.pallas import tpu as pltpu

_LANE = 128
_SUB = 8
_VMEM = 64 * 1024 * 1024


def _ceil_to(v, m):
    return ((v + m - 1) // m) * m


def _maybe_pad(a, rows, cols):
    if a.shape == (rows, cols):
        return a
    return jnp.pad(a, ((0, rows - a.shape[0]), (0, cols - a.shape[1])))


def _dgi_body(x_ref, xc_ref, w_ref, b_ref, a_ref, wp_ref, bp_ref,
              z_ref, zn_ref, g_ref, xs_scr, cs_scr, *,
              dcols, n_real, band, scale):
    i = pl.program_id(0)

    @pl.when(i == 0)
    def _stage():
        xs_scr[:, :dcols] = x_ref[...].astype(jnp.bfloat16)
        xs_scr[:, dcols:] = xc_ref[...].astype(jnp.bfloat16)
        cs_scr[...] = jnp.zeros_like(cs_scr)

    t = jnp.dot(a_ref[...], xs_scr[...], preferred_element_type=jnp.float32)
    w = w_ref[...]
    bias = b_ref[...]
    zc = jnp.dot(t[:, :dcols], w, preferred_element_type=jnp.float32) + bias
    zn = jnp.dot(t[:, dcols:], w, preferred_element_type=jnp.float32) + bias
    if n_real is not None:
        rows = band * i + jax.lax.broadcasted_iota(jnp.int32, zc.shape, 0)
        zc = jnp.where(rows < n_real, zc, 0.0)
        zn = jnp.where(rows < n_real, zn, 0.0)
    z_ref[...] = zc
    zn_ref[...] = zn
    cs_scr[...] += jnp.broadcast_to(
        jnp.sum(zc, axis=0, keepdims=True), cs_scr.shape
    )

    @pl.when(i == pl.num_programs(0) - 1)
    def _readout():
        s = jax.nn.sigmoid(
            jnp.sum(cs_scr[...], axis=0, keepdims=True) * scale
        )
        g_ref[...] = bp_ref[...] + jax.lax.dot_general(
            s, wp_ref[...], (((1,), (1,)), ((), ())),
            preferred_element_type=jnp.float32,
        )


def kernel(x, x_corrupt, a_pad, w_enc, b_enc, w_proj, b_proj):
    n, d_in = x.shape
    hdim = w_enc.shape[1]
    n_pad = a_pad.shape[0]
    d_pad = _ceil_to(d_in, _LANE)
    h_pad = _ceil_to(hdim, _LANE)

    x_p = _maybe_pad(x, n_pad, d_pad)
    xc_p = _maybe_pad(x_corrupt, n_pad, d_pad)
    w_p = _maybe_pad(w_enc, d_pad, h_pad).astype(jnp.float32)
    be_p = _maybe_pad(b_enc, 1, h_pad).astype(jnp.float32)
    wp_p = _maybe_pad(w_proj, h_pad, h_pad).astype(jnp.float32)
    bp_p = _maybe_pad(b_proj, 1, h_pad).astype(jnp.float32)

    band = min(256, n_pad)
    nba = n_pad // band

    z_p, zn_p, g_p = pl.pallas_call(
        functools.partial(
            _dgi_body, dcols=d_pad, n_real=None if n == n_pad else n,
            band=band, scale=1.0 / (_SUB * n),
        ),
        out_shape=(
            jax.ShapeDtypeStruct((n_pad, h_pad), jnp.float32),
            jax.ShapeDtypeStruct((n_pad, h_pad), jnp.float32),
            jax.ShapeDtypeStruct((1, h_pad), jnp.float32),
        ),
        grid=(nba,),
        in_specs=[
            pl.BlockSpec((n_pad, d_pad), lambda i: (0, 0)),
            pl.BlockSpec((n_pad, d_pad), lambda i: (0, 0)),
            pl.BlockSpec((d_pad, h_pad), lambda i: (0, 0)),
            pl.BlockSpec((1, h_pad), lambda i: (0, 0)),
            pl.BlockSpec((band, n_pad), lambda i: (i, 0)),
            pl.BlockSpec((h_pad, h_pad), lambda i: (0, 0)),
            pl.BlockSpec((1, h_pad), lambda i: (0, 0)),
        ],
        out_specs=[
            pl.BlockSpec((band, h_pad), lambda i: (i, 0)),
            pl.BlockSpec((band, h_pad), lambda i: (i, 0)),
            pl.BlockSpec((1, h_pad), lambda i: (0, 0)),
        ],
        scratch_shapes=[
            pltpu.VMEM((n_pad, 2 * d_pad), jnp.bfloat16),
            pltpu.VMEM((_SUB, h_pad), jnp.float32),
        ],
        compiler_params=pltpu.CompilerParams(
            dimension_semantics=("arbitrary",),
            vmem_limit_bytes=_VMEM,
        ),
        cost_estimate=pl.CostEstimate(
            flops=4 * n_pad * n_pad * d_pad + 8 * n_pad * d_pad * h_pad,
            transcendentals=h_pad,
            bytes_accessed=n_pad * n_pad * 2
            + 2 * n_pad * d_pad * 4
            + 2 * n_pad * h_pad * 4,
        ),
    )(x_p, xc_p, w_p, be_p, a_pad, wp_p, bp_p)

    z = z_p if (n, hdim) == (n_pad, h_pad) else z_p[:n, :hdim]
    zn = zn_p if (n, hdim) == (n_pad, h_pad) else zn_p[:n, :hdim]
    g = g_p if hdim == h_pad else g_p[:, :hdim]
    return z, g, zn


# speedup vs baseline: 1.1599x; 1.1599x over previous
import functools

import jax
import jax.numpy as jnp
from jax.experimental import pallas as pl
from jax.experimental.pallas import tpu as pltpu

_LANE = 128
_SUB = 8
_VMEM = 64 * 1024 * 1024


def _ceil_to(v, m):
    return ((v + m - 1) // m) * m


def _maybe_pad(a, rows, cols):
    if a.shape == (rows, cols):
        return a
    return jnp.pad(a, ((0, rows - a.shape[0]), (0, cols - a.shape[1])))


def _dgi_body(x_ref, xc_ref, w_ref, b_ref, a_ref, wp_ref, bp_ref,
              z_ref, zn_ref, g_ref, xs_scr, cs_scr, *,
              dcols, n_real, band, scale):
    i = pl.program_id(0)

    @pl.when(i == 0)
    def _stage():
        xs_scr[:, :dcols] = x_ref[...].astype(jnp.bfloat16)
        xs_scr[:, dcols:] = xc_ref[...].astype(jnp.bfloat16)
        cs_scr[...] = jnp.zeros_like(cs_scr)

    t = jnp.dot(a_ref[...], xs_scr[...], preferred_element_type=jnp.float32)
    w = w_ref[...]
    bias = b_ref[...]
    zc = jnp.dot(t[:, :dcols], w, preferred_element_type=jnp.float32) + bias
    zn = jnp.dot(t[:, dcols:], w, preferred_element_type=jnp.float32) + bias
    if n_real is not None:
        rows = band * i + jax.lax.broadcasted_iota(jnp.int32, zc.shape, 0)
        zc = jnp.where(rows < n_real, zc, 0.0)
        zn = jnp.where(rows < n_real, zn, 0.0)
    z_ref[...] = zc
    zn_ref[...] = zn
    cs_scr[...] += jnp.broadcast_to(
        jnp.sum(zc, axis=0, keepdims=True), cs_scr.shape
    )

    @pl.when(i == pl.num_programs(0) - 1)
    def _readout():
        s = jax.nn.sigmoid(
            jnp.sum(cs_scr[...], axis=0, keepdims=True) * scale
        )
        g_ref[...] = bp_ref[...] + jax.lax.dot_general(
            s, wp_ref[...], (((1,), (1,)), ((), ())),
            preferred_element_type=jnp.float32,
        )


def kernel(x, x_corrupt, a_pad, w_enc, b_enc, w_proj, b_proj):
    n, d_in = x.shape
    hdim = w_enc.shape[1]
    n_pad = a_pad.shape[0]
    d_pad = _ceil_to(d_in, _LANE)
    h_pad = _ceil_to(hdim, _LANE)

    x_p = _maybe_pad(x, n_pad, d_pad)
    xc_p = _maybe_pad(x_corrupt, n_pad, d_pad)
    w_p = _maybe_pad(w_enc, d_pad, h_pad).astype(jnp.float32)
    be_p = _maybe_pad(b_enc, 1, h_pad).astype(jnp.float32)
    wp_p = _maybe_pad(w_proj, h_pad, h_pad).astype(jnp.float32)
    bp_p = _maybe_pad(b_proj, 1, h_pad).astype(jnp.float32)

    band = min(1024, n_pad)
    nba = n_pad // band

    z_p, zn_p, g_p = pl.pallas_call(
        functools.partial(
            _dgi_body, dcols=d_pad, n_real=None if n == n_pad else n,
            band=band, scale=1.0 / (_SUB * n),
        ),
        out_shape=(
            jax.ShapeDtypeStruct((n_pad, h_pad), jnp.float32),
            jax.ShapeDtypeStruct((n_pad, h_pad), jnp.float32),
            jax.ShapeDtypeStruct((1, h_pad), jnp.float32),
        ),
        grid=(nba,),
        in_specs=[
            pl.BlockSpec((n_pad, d_pad), lambda i: (0, 0)),
            pl.BlockSpec((n_pad, d_pad), lambda i: (0, 0)),
            pl.BlockSpec((d_pad, h_pad), lambda i: (0, 0)),
            pl.BlockSpec((1, h_pad), lambda i: (0, 0)),
            pl.BlockSpec((band, n_pad), lambda i: (i, 0)),
            pl.BlockSpec((h_pad, h_pad), lambda i: (0, 0)),
            pl.BlockSpec((1, h_pad), lambda i: (0, 0)),
        ],
        out_specs=[
            pl.BlockSpec((band, h_pad), lambda i: (i, 0)),
            pl.BlockSpec((band, h_pad), lambda i: (i, 0)),
            pl.BlockSpec((1, h_pad), lambda i: (0, 0)),
        ],
        scratch_shapes=[
            pltpu.VMEM((n_pad, 2 * d_pad), jnp.bfloat16),
            pltpu.VMEM((_SUB, h_pad), jnp.float32),
        ],
        compiler_params=pltpu.CompilerParams(
            dimension_semantics=("arbitrary",),
            vmem_limit_bytes=_VMEM,
        ),
        cost_estimate=pl.CostEstimate(
            flops=4 * n_pad * n_pad * d_pad + 8 * n_pad * d_pad * h_pad,
            transcendentals=h_pad,
            bytes_accessed=n_pad * n_pad * 2
            + 2 * n_pad * d_pad * 4
            + 2 * n_pad * h_pad * 4,
        ),
    )(x_p, xc_p, w_p, be_p, a_pad, wp_p, bp_p)

    z = z_p if (n, hdim) == (n_pad, h_pad) else z_p[:n, :hdim]
    zn = zn_p if (n, hdim) == (n_pad, h_pad) else zn_p[:n, :hdim]
    g = g_p if hdim == h_pad else g_p[:, :hdim]
    return z, g, zn
